# baseline (speedup 1.0000x reference)
import functools

import jax
import jax.numpy as jnp
from jax import lax
from jax.experimental import pallas as pl
from jax.experimental.pallas import tpu as pltpu
from jax.experimental.pallas import tpu_sc as plsc

_MAX_SEQUENCE_LENGTH = 10000

_NC, _NS, _L = 2, 16, 16
_NW = _NC * _NS


def _positional_encoding(seq_len, d_model):
    position = jnp.arange(seq_len, dtype=jnp.float32).reshape(seq_len, 1)
    dim = jnp.floor_divide(jnp.linspace(0.0, d_model - 1, d_model), 2.0) * 2.0
    dim = dim / d_model
    denom = jnp.power(jnp.float32(_MAX_SEQUENCE_LENGTH), dim)
    angles = position / denom
    col = jnp.arange(d_model)
    return jnp.where(col % 2 == 0, jnp.sin(angles), jnp.cos(angles)).astype(jnp.float32)


@functools.cache
def _build(B, S, D):
    N = B * S
    chunk = B // _NW
    nchunk = S
    rows_per_w = chunk * nchunk
    assert B % _NW == 0 and chunk <= 128 and chunk % _L == 0
    assert D % _L == 0
    assert nchunk % 3 == 2 and nchunk >= 8

    mesh = plsc.VectorSubcoreMesh(core_axis_name="c", subcore_axis_name="s")

    rows_t = pltpu.VMEM((chunk, D), jnp.float32)
    oidx_t = pltpu.VMEM((chunk,), jnp.int32)

    @functools.partial(
        pl.kernel,
        mesh=mesh,
        out_type=jax.ShapeDtypeStruct((N, D), jnp.float32),
        scratch_types=[
            pltpu.VMEM((rows_per_w,), jnp.int32),
            pltpu.VMEM((S, D), jnp.float32),
            rows_t, rows_t, rows_t,
            oidx_t, oidx_t, oidx_t,
            pltpu.SemaphoreType.DMA, pltpu.SemaphoreType.DMA,
            pltpu.SemaphoreType.DMA, pltpu.SemaphoreType.DMA,
            pltpu.SemaphoreType.DMA, pltpu.SemaphoreType.DMA,
        ],
    )
    def emb(idx_hbm, table_hbm, pe_hbm, out_hbm,
            idx_v, pe_v, rb0, rb1, rb2, ox0, ox1, ox2, g0, g1, g2, s0, s1, s2):
        wid = lax.axis_index("s") * _NC + lax.axis_index("c")
        base = wid * rows_per_w
        pltpu.sync_copy(idx_hbm.at[pl.ds(base, rows_per_w)], idx_v)
        pltpu.sync_copy(pe_hbm, pe_v)

        bufs = (rb0, rb1, rb2)
        oxs = (ox0, ox1, ox2)
        gsems = (g0, g1, g2)
        ssems = (s0, s1, s2)
        jbase = wid * chunk * S

        def gcopy(c, b):
            return pltpu.make_async_copy(
                table_hbm.at[idx_v.at[pl.ds(c * chunk, chunk)]], bufs[b], gsems[b])

        half = chunk // 2

        def gstart2(c, b):
            pltpu.make_async_copy(
                table_hbm.at[idx_v.at[pl.ds(c * chunk, half)]],
                bufs[b].at[pl.ds(0, half)], gsems[b]).start()
            pltpu.make_async_copy(
                table_hbm.at[idx_v.at[pl.ds(c * chunk + half, half)]],
                bufs[b].at[pl.ds(half, half)], gsems[b]).start()

        def scopy(b):
            return pltpu.make_async_copy(bufs[b], out_hbm.at[oxs[b]], ssems[b])

        def add_pe_and_oidx(c, b):
            buf, ox = bufs[b], oxs[b]
            obase = jbase + c
            for g in range(chunk // _L):
                lane = lax.iota(jnp.int32, _L) + (g * _L)
                ox[pl.ds(g * _L, _L)] = lane * S + obase
            pe_regs = [pe_v[c, pl.ds(g * _L, _L)] for g in range(D // _L)]

            @plsc.parallel_loop(0, chunk, step=1, unroll=2)
            def _(j):
                for g in range(D // _L):
                    plsc.addupdate(buf.at[j, pl.ds(g * _L, _L)], pe_regs[g])

        def step(c, b, *, swait_prev=True, gstart_ahead=True):
            gcopy(c, b).wait()
            add_pe_and_oidx(c, b)
            scopy(b).start()
            if swait_prev:
                scopy((b - 1) % 3).wait()
            if gstart_ahead:
                gstart2(c + 2, (b + 2) % 3)

        gstart2(0, 0)
        gstart2(1, 1)
        step(0, 0, swait_prev=False)
        step(1, 1)
        step(2, 2)

        def mid(p, carry):
            c = p * 3
            for u in range(3):
                step(c + u, u)
            return carry

        lax.fori_loop(1, nchunk // 3, mid, 0)

        step(nchunk - 2, 0, gstart_ahead=False)
        step(nchunk - 1, 1, gstart_ahead=False)
        scopy(1).wait()

    return emb


def kernel(x, token_embedding_weight):
    B, S = x.shape
    _, D = token_embedding_weight.shape
    pe = _positional_encoding(S, D)
    idx = (x.reshape(_NW, B // _NW, S)
             .transpose(0, 2, 1)
             .reshape(-1)
             .astype(jnp.int32))
    out = _build(B, S, D)(idx, token_embedding_weight, pe)
    return out.reshape(B, S, D)
